# baseline (speedup 1.0000x reference)
import functools

import jax
import jax.numpy as jnp
from jax import lax
from jax.experimental import pallas as pl
from jax.experimental.pallas import tpu as pltpu
from jax.experimental.pallas import tpu_sc as plsc

N_USERS = 1000000
N_ITEMS = 1000000
F = 32
B = 16384
H = 8

_NC = 2
_NS = 16
_NW = _NC * _NS
_BPW = B // _NW
_CH = 128
_NCH = _BPW // _CH


def _gather_body(user_hbm, item_hbm, uf_hbm, if_hbm, ue_out, ie_out,
                 uidx_v, iidx_v, ue_v, ie_v, sem):
    wid = lax.axis_index("s") * _NC + lax.axis_index("c")
    base = wid * _BPW
    pltpu.sync_copy(user_hbm.at[pl.ds(base, _BPW)], uidx_v)
    pltpu.sync_copy(item_hbm.at[pl.ds(base, _BPW)], iidx_v)
    copies = []
    for c in range(_NCH):
        sl = pl.ds(c * _CH, _CH)
        copies.append(pltpu.async_copy(uf_hbm.at[uidx_v.at[sl]], ue_v.at[sl], sem))
        copies.append(pltpu.async_copy(if_hbm.at[iidx_v.at[sl]], ie_v.at[sl], sem))
    for cp in copies:
        cp.wait()
    pltpu.sync_copy(ue_v, ue_out.at[pl.ds(base, _BPW)])
    pltpu.sync_copy(ie_v, ie_out.at[pl.ds(base, _BPW)])


_sc_gather = functools.partial(
    pl.kernel,
    out_type=[
        jax.ShapeDtypeStruct((B, F), jnp.float32),
        jax.ShapeDtypeStruct((B, F), jnp.float32),
    ],
    mesh=plsc.VectorSubcoreMesh(core_axis_name="c", subcore_axis_name="s"),
    scratch_types=[
        pltpu.VMEM((_BPW,), jnp.int32),
        pltpu.VMEM((_BPW,), jnp.int32),
        pltpu.VMEM((_BPW, F), jnp.float32),
        pltpu.VMEM((_BPW, F), jnp.float32),
        pltpu.SemaphoreType.DMA,
    ],
    compiler_params=pltpu.CompilerParams(use_tc_tiling_on_sc=False),
)(_gather_body)


def _mlp_body(ue_ref, ie_ref, w1u_ref, w1i_ref, b1_ref, w3_ref, b3_ref, out_ref):
    h = (jnp.dot(ue_ref[...], w1u_ref[...], preferred_element_type=jnp.float32)
         + jnp.dot(ie_ref[...], w1i_ref[...], preferred_element_type=jnp.float32)
         + b1_ref[...])
    h = jnp.maximum(h, 0.0)
    z = jnp.dot(h, w3_ref[...], preferred_element_type=jnp.float32) + b3_ref[...]
    out_ref[...] = jax.nn.sigmoid(z)


_BLK = 4096


def _mlp(ue, ie, w1u, w1i, b1, w3, b3):
    grid = (B // _BLK,)
    return pl.pallas_call(
        _mlp_body,
        grid=grid,
        in_specs=[
            pl.BlockSpec((_BLK, F), lambda i: (i, 0)),
            pl.BlockSpec((_BLK, F), lambda i: (i, 0)),
            pl.BlockSpec((F, H), lambda i: (0, 0)),
            pl.BlockSpec((F, H), lambda i: (0, 0)),
            pl.BlockSpec((1, H), lambda i: (0, 0)),
            pl.BlockSpec((H, 1), lambda i: (0, 0)),
            pl.BlockSpec((1, 1), lambda i: (0, 0)),
        ],
        out_specs=pl.BlockSpec((_BLK, 1), lambda i: (i, 0)),
        out_shape=jax.ShapeDtypeStruct((B, 1), jnp.float32),
    )(ue, ie, w1u, w1i, b1, w3, b3)


def kernel(user, item, user_factors, item_factors, W1, b1, W3, b3):
    user = user.astype(jnp.int32)
    item = item.astype(jnp.int32)
    ue, ie = _sc_gather(user, item, user_factors, item_factors)
    return _mlp(ue, ie, W1[:F], W1[F:], b1.reshape(1, H), W3, b3.reshape(1, 1))


# speedup vs baseline: 2.2066x; 2.2066x over previous
import functools

import jax
import jax.numpy as jnp
from jax import lax
from jax.experimental import pallas as pl
from jax.experimental.pallas import tpu as pltpu
from jax.experimental.pallas import tpu_sc as plsc

N_ROWS = 1000000
F = 32
B = 16384
H = 8

_NC = 2
_NS = 16
_NW = _NC * _NS
_BPW = B // _NW
_CH = 32
_NCHUNK = _BPW // _CH
_TILES = N_ROWS // 8
_L = 16


def _extract_rows(buf, idx_v, stage, chunk_off):
    lanes = lax.iota(jnp.int32, _L)
    for g in range(_CH // _L):
        sub_vec = jnp.bitwise_and(idx_v[pl.ds(chunk_off + g * _L, _L)], 7)
        for r in range(_L):
            row = g * _L + r
            sub = jnp.full((_L,), sub_vec[r], jnp.int32)
            rr = jnp.full((_L,), row, jnp.int32)
            lo = plsc.load_gather(buf, [rr, sub, lanes])
            hi = plsc.load_gather(buf, [rr, sub, lanes + _L])
            stage[row, pl.ds(0, _L)] = lo
            stage[row, pl.ds(_L, _L)] = hi


def _gather_body(user_hbm, item_hbm, uf_hbm, if_hbm, ue_out, ie_out,
                 uidx_s, iidx_s, ubuf, ibuf, ustage, istage, sem):
    wid = lax.axis_index("s") * _NC + lax.axis_index("c")
    base = wid * _BPW
    pltpu.sync_copy(user_hbm.at[pl.ds(base, _BPW)], uidx_s)
    pltpu.sync_copy(item_hbm.at[pl.ds(base, _BPW)], iidx_s)

    def chunk_body(k, _):
        off = k * _CH
        copies = []
        for g in range(_CH // _L):
            ut_vec = jnp.right_shift(uidx_s[pl.ds(off + g * _L, _L)], 3)
            it_vec = jnp.right_shift(iidx_s[pl.ds(off + g * _L, _L)], 3)
            for r in range(_L):
                row = g * _L + r
                copies.append(
                    pltpu.async_copy(uf_hbm.at[ut_vec[r]], ubuf.at[row], sem))
                copies.append(
                    pltpu.async_copy(if_hbm.at[it_vec[r]], ibuf.at[row], sem))
        for cp in copies:
            cp.wait()
        _extract_rows(ubuf, uidx_s, ustage, off)
        _extract_rows(ibuf, iidx_s, istage, off)
        pltpu.sync_copy(ustage, ue_out.at[pl.ds(base + off, _CH)])
        pltpu.sync_copy(istage, ie_out.at[pl.ds(base + off, _CH)])
        return ()

    lax.fori_loop(0, _NCHUNK, chunk_body, (), unroll=False)


_sc_gather = functools.partial(
    pl.kernel,
    out_type=[
        jax.ShapeDtypeStruct((B, F), jnp.float32),
        jax.ShapeDtypeStruct((B, F), jnp.float32),
    ],
    mesh=plsc.VectorSubcoreMesh(core_axis_name="c", subcore_axis_name="s"),
    scratch_types=[
        pltpu.VMEM((_BPW,), jnp.int32),
        pltpu.VMEM((_BPW,), jnp.int32),
        pltpu.VMEM((_CH, 8, F), jnp.float32),
        pltpu.VMEM((_CH, 8, F), jnp.float32),
        pltpu.VMEM((_CH, F), jnp.float32),
        pltpu.VMEM((_CH, F), jnp.float32),
        pltpu.SemaphoreType.DMA,
    ],
    compiler_params=pltpu.CompilerParams(needs_layout_passes=False),
)(_gather_body)


def _mlp_body(ue_ref, ie_ref, w1u_ref, w1i_ref, b1_ref, w3_ref, b3_ref, out_ref):
    h = (jnp.dot(ue_ref[...], w1u_ref[...], preferred_element_type=jnp.float32)
         + jnp.dot(ie_ref[...], w1i_ref[...], preferred_element_type=jnp.float32)
         + b1_ref[...])
    h = jnp.maximum(h, 0.0)
    z = jnp.dot(h, w3_ref[...], preferred_element_type=jnp.float32) + b3_ref[...]
    out_ref[...] = jax.nn.sigmoid(z)


_BLK = 4096


def _mlp(ue, ie, w1u, w1i, b1, w3, b3):
    grid = (B // _BLK,)
    return pl.pallas_call(
        _mlp_body,
        grid=grid,
        in_specs=[
            pl.BlockSpec((_BLK, F), lambda i: (i, 0)),
            pl.BlockSpec((_BLK, F), lambda i: (i, 0)),
            pl.BlockSpec((F, H), lambda i: (0, 0)),
            pl.BlockSpec((F, H), lambda i: (0, 0)),
            pl.BlockSpec((1, H), lambda i: (0, 0)),
            pl.BlockSpec((H, 1), lambda i: (0, 0)),
            pl.BlockSpec((1, 1), lambda i: (0, 0)),
        ],
        out_specs=pl.BlockSpec((_BLK, 1), lambda i: (i, 0)),
        out_shape=jax.ShapeDtypeStruct((B, 1), jnp.float32),
    )(ue, ie, w1u, w1i, b1, w3, b3)


def kernel(user, item, user_factors, item_factors, W1, b1, W3, b3):
    user = user.astype(jnp.int32)
    item = item.astype(jnp.int32)
    uf3 = user_factors.reshape(_TILES, 8, F)
    if3 = item_factors.reshape(_TILES, 8, F)
    ue, ie = _sc_gather(user, item, uf3, if3)
    return _mlp(ue, ie, W1[:F], W1[F:], b1.reshape(1, H), W3, b3.reshape(1, 1))
